# Initial kernel scaffold; baseline (speedup 1.0000x reference)
import jax
import jax.numpy as jnp
from jax.experimental import pallas as pl


def kernel(x, edge_index, edge_attr, batch, target, params):
    raise NotImplementedError("write your pallas kernel here")


# speedup vs baseline: 1.3227x; 1.3227x over previous
import functools

import jax
import jax.numpy as jnp
from jax import lax
from jax.experimental import pallas as pl
from jax.experimental.pallas import tpu as pltpu
from jax.experimental.pallas import tpu_sc as plsc

_N = 50000
_E = 800000
_NG = 512
_L = 1000
_DIM = 32
_NFX = 78
_OUT_DIM = 128

_NC, _NS = 2, 16
_NW = _NC * _NS

_CHUNK = 256
_SUB = 2
_E_PAD = 819200
_CHUNKS_PW = _E_PAD // (_NW * _CHUNK)
_N_PAD = 51200
_ROWS_PT = _N_PAD // _NS
_ZCHUNKS = _ROWS_PT // 128

_N_PAD2 = 53248
_PCH = _N_PAD2 // (_NW * 128)

_NEG = -1e30
_INV_STD = 1.0 / (1.0 + 1e-05) ** 0.5


def _edge_pass_body(h_hbm, ea_hbm, src_hbm, dst_hbm, out_hbm,
                    src_v, dst_v, rows_v, ea_v, zero_v, agg_sh, sem):
    cid = lax.axis_index("c")
    sid = lax.axis_index("s")
    wid = cid * _NS + sid

    @pl.loop(0, 128)
    def _zfill(r):
        zero_v[r, pl.ds(0, 16)] = jnp.zeros((16,), jnp.float32)
        zero_v[r, pl.ds(16, 16)] = jnp.zeros((16,), jnp.float32)

    tbase = sid * _ROWS_PT

    @pl.loop(0, _ZCHUNKS)
    def _zcopy(t):
        pltpu.sync_copy(zero_v, agg_sh.at[pl.ds(tbase + t * 128, 128)])

    plsc.subcore_barrier()

    ebase_blk = wid * (_CHUNKS_PW * _SUB)

    @pl.loop(0, _CHUNKS_PW)
    def _chunk(tc):
        blk = ebase_blk + tc * _SUB
        pltpu.sync_copy(src_hbm.at[pl.ds(blk, _SUB)], src_v)
        pltpu.sync_copy(dst_hbm.at[pl.ds(blk, _SUB)], dst_v)
        pltpu.sync_copy(ea_hbm.at[pl.ds(blk * 128, _CHUNK)], ea_v)
        descs = [
            pltpu.async_copy(h_hbm.at[src_v.at[j]],
                             rows_v.at[pl.ds(j * 128, 128)], sem)
            for j in range(_SUB)
        ]
        for d in descs:
            d.wait()

        @plsc.parallel_loop(0, _CHUNK, unroll=8)
        def _relu_add(r):
            a0 = rows_v[r, pl.ds(0, 16)] + ea_v[r, pl.ds(0, 16)]
            rows_v[r, pl.ds(0, 16)] = jnp.maximum(a0, 0.0)
            a1 = rows_v[r, pl.ds(16, 16)] + ea_v[r, pl.ds(16, 16)]
            rows_v[r, pl.ds(16, 16)] = jnp.maximum(a1, 0.0)

        for j in range(_SUB):
            pltpu.sync_copy(rows_v.at[pl.ds(j * 128, 128)],
                            agg_sh.at[dst_v.at[j]], add=True)

    plsc.subcore_barrier()

    @pl.loop(0, _ZCHUNKS)
    def _ocopy(t):
        pltpu.sync_copy(agg_sh.at[pl.ds(tbase + t * 128, 128)],
                        out_hbm.at[cid, pl.ds(tbase + t * 128, 128)])


def _edge_pass(h, ea_slab, src_r, dst_r):
    mesh = plsc.VectorSubcoreMesh(core_axis_name="c", subcore_axis_name="s",
                                  num_cores=_NC, num_subcores=_NS)
    return pl.kernel(
        _edge_pass_body,
        out_type=jax.ShapeDtypeStruct((_NC, _N_PAD, _DIM), jnp.float32),
        mesh=mesh,
        scratch_types=[
            pltpu.VMEM((_SUB, 128), jnp.int32),
            pltpu.VMEM((_SUB, 128), jnp.int32),
            pltpu.VMEM((_CHUNK, _DIM), jnp.float32),
            pltpu.VMEM((_CHUNK, _DIM), jnp.float32),
            pltpu.VMEM((128, _DIM), jnp.float32),
            pltpu.VMEM_SHARED((_N_PAD, _DIM), jnp.float32),
            pltpu.SemaphoreType.DMA,
        ],
        compiler_params=pltpu.CompilerParams(use_tc_tiling_on_sc=False),
    )(h, ea_slab, src_r, dst_r)


def _pool_body(h_hbm, b_hbm, out_hbm, idx_v, hbuf_v, zero_v, pool_sh):
    cid = lax.axis_index("c")
    sid = lax.axis_index("s")
    wid = cid * _NS + sid

    @pl.loop(0, 32)
    def _zfill(r):
        zero_v[r, pl.ds(0, 16)] = jnp.zeros((16,), jnp.float32)
        zero_v[r, pl.ds(16, 16)] = jnp.zeros((16,), jnp.float32)

    pltpu.sync_copy(zero_v, pool_sh.at[pl.ds(sid * 32, 32)])
    plsc.subcore_barrier()

    pltpu.sync_copy(b_hbm.at[pl.ds(wid * _PCH, _PCH)], idx_v)

    @pl.loop(0, _PCH)
    def _chunk(t):
        pltpu.sync_copy(h_hbm.at[pl.ds((wid * _PCH + t) * 128, 128)], hbuf_v)
        pltpu.sync_copy(hbuf_v, pool_sh.at[idx_v.at[t]], add=True)

    plsc.subcore_barrier()
    pltpu.sync_copy(pool_sh.at[pl.ds(sid * 32, 32)],
                    out_hbm.at[cid, pl.ds(sid * 32, 32)])


def _pool_pass(h_pad, batch_r):
    mesh = plsc.VectorSubcoreMesh(core_axis_name="c", subcore_axis_name="s",
                                  num_cores=_NC, num_subcores=_NS)
    return pl.kernel(
        _pool_body,
        out_type=jax.ShapeDtypeStruct((_NC, _NG, _DIM), jnp.float32),
        mesh=mesh,
        scratch_types=[
            pltpu.VMEM((_PCH, 128), jnp.int32),
            pltpu.VMEM((128, _DIM), jnp.float32),
            pltpu.VMEM((32, _DIM), jnp.float32),
            pltpu.VMEM_SHARED((_NG, _DIM), jnp.float32),
        ],
        compiler_params=pltpu.CompilerParams(use_tc_tiling_on_sc=False),
    )(h_pad, batch_r)


_BE = 2048


def _eproj_body(ea_ref, eew, eeb, w1, b1, w2, b2, w3, b3, w4, b4, w5, b5,
                o1a, o1b, o1c, o2, o3, o4, o5):
    i = pl.program_id(0)
    a = ea_ref[...]
    e = jnp.dot(a, eew[...], preferred_element_type=jnp.float32) + eeb[...]
    row = i * _BE + lax.broadcasted_iota(jnp.int32, (_BE, 1), 0)
    valid = row < _E

    def emit(o_ref, w, b):
        v = jnp.dot(e, w, preferred_element_type=jnp.float32) + b
        o_ref[...] = jnp.where(valid, v, _NEG)

    w1f, b1f = w1[...], b1[...]
    emit(o1a, w1f[:, 0:32], b1f[:, 0:32])
    emit(o1b, w1f[:, 32:64], b1f[:, 32:64])
    emit(o1c, w1f[:, 64:96], b1f[:, 64:96])
    emit(o2, w2[...], b2[...])
    emit(o3, w3[...], b3[...])
    emit(o4, w4[...], b4[...])
    emit(o5, w5[...], b5[...])


def _eproj(ea_pad, eew, eeb, ws, bs):
    full = lambda shape: pl.BlockSpec(shape, lambda i: (0,) * len(shape))
    out32 = jax.ShapeDtypeStruct((_E_PAD, _DIM), jnp.float32)
    in_specs = [pl.BlockSpec((_BE, 7), lambda i: (i, 0)),
                full((7, 32)), full((1, 32)),
                full((32, 96)), full((1, 96))]
    for _ in range(4):
        in_specs += [full((32, 32)), full((1, 32))]
    return pl.pallas_call(
        _eproj_body,
        grid=(_E_PAD // _BE,),
        in_specs=in_specs,
        out_specs=[pl.BlockSpec((_BE, _DIM), lambda i: (i, 0))] * 7,
        out_shape=[out32] * 7,
    )(ea_pad, eew, eeb, ws[0], bs[0], ws[1], bs[1], ws[2], bs[2],
      ws[3], bs[3], ws[4], bs[4])


_BN = 2000


def _mlp_tail(acc, w2, b2, g, bb, o_ref):
    t = jnp.maximum(acc, 0.0)
    z2 = jnp.dot(t, w2[...], preferred_element_type=jnp.float32) + b2[...]
    o_ref[...] = jnp.maximum(z2, 0.0) * g[...] + bb[...]


def _mlp1_body(x_ref, a0, a1, a2, w1, b1, w2, b2, g, bb, o_ref):
    xf = x_ref[...]
    w1f = w1[...]
    acc = jnp.broadcast_to(b1[...], (_BN, _DIM)).astype(jnp.float32)
    for j, ar in enumerate((a0, a1, a2)):
        zj = xf[:, 32 * j:32 * j + 32] + ar[0] + ar[1]
        acc = acc + jnp.dot(zj, w1f[32 * j:32 * j + 32, :],
                            preferred_element_type=jnp.float32)
    _mlp_tail(acc, w2, b2, g, bb, o_ref)


def _mlp_body(h_ref, ag, w1, b1, w2, b2, g, bb, o_ref):
    z = h_ref[...] + ag[0] + ag[1]
    acc = jnp.dot(z, w1[...], preferred_element_type=jnp.float32) + b1[...]
    _mlp_tail(acc, w2, b2, g, bb, o_ref)


def _node_mlp1(x_pad, aggs, w1p, b1, w2, b2, g, bb):
    full = lambda shape: pl.BlockSpec(shape, lambda i: (0,) * len(shape))
    aspec = pl.BlockSpec((_NC, _BN, _DIM), lambda i: (0, i, 0))
    return pl.pallas_call(
        _mlp1_body,
        grid=(_N // _BN,),
        in_specs=[pl.BlockSpec((_BN, 96), lambda i: (i, 0)),
                  aspec, aspec, aspec,
                  full((96, 32)), full((1, 32)), full((32, 32)),
                  full((1, 32)), full((1, 32)), full((1, 32))],
        out_specs=pl.BlockSpec((_BN, _DIM), lambda i: (i, 0)),
        out_shape=jax.ShapeDtypeStruct((_N, _DIM), jnp.float32),
    )(x_pad, aggs[0], aggs[1], aggs[2], w1p, b1, w2, b2, g, bb)


def _node_mlp(h, agg, w1, b1, w2, b2, g, bb):
    full = lambda shape: pl.BlockSpec(shape, lambda i: (0,) * len(shape))
    return pl.pallas_call(
        _mlp_body,
        grid=(_N // _BN,),
        in_specs=[pl.BlockSpec((_BN, _DIM), lambda i: (i, 0)),
                  pl.BlockSpec((_NC, _BN, _DIM), lambda i: (0, i, 0)),
                  full((32, 32)), full((1, 32)), full((32, 32)),
                  full((1, 32)), full((1, 32)), full((1, 32))],
        out_specs=pl.BlockSpec((_BN, _DIM), lambda i: (i, 0)),
        out_shape=jax.ShapeDtypeStruct((_N, _DIM), jnp.float32),
    )(h, agg, w1, b1, w2, b2, g, bb)


_GB = 8


def _prot_body(tg_ref, emb, cw, cb, fw, fb, o_ref):
    embf = emb[...]
    cwf = cw[...]
    tf = tg_ref[...]
    for gi in range(_GB):
        oh = (lax.broadcasted_iota(jnp.int32, (26, _L), 0)
              == tf[gi:gi + 1, :]).astype(jnp.float32)
        rows = lax.dot_general(oh, embf, (((0,), (0,)), ((), ())),
                               preferred_element_type=jnp.float32,
                               precision=lax.Precision.HIGHEST)
        acc = jnp.zeros((_L - 7, _DIM), jnp.float32)
        for k in range(8):
            acc = acc + jnp.dot(rows[k:k + _L - 7, :], cwf[k],
                                preferred_element_type=jnp.float32,
                                precision=lax.Precision.HIGHEST)
        mx = jnp.max(acc, axis=0, keepdims=True)
        xt1 = jnp.maximum(mx + cb[...], 0.0)
        o_ref[gi:gi + 1, :] = jnp.maximum(
            jnp.dot(xt1, fw[...], preferred_element_type=jnp.float32) + fb[...], 0.0)


def _protein(target, emb, cw, cb, fw, fb):
    full = lambda shape: pl.BlockSpec(shape, lambda i: (0,) * len(shape))
    return pl.pallas_call(
        _prot_body,
        grid=(_NG // _GB,),
        in_specs=[pl.BlockSpec((_GB, _L), lambda i: (i, 0)),
                  full((26, 128)), full((8, 128, 32)), full((1, 32)),
                  full((32, 128)), full((1, 128))],
        out_specs=pl.BlockSpec((_GB, _OUT_DIM), lambda i: (i, 0)),
        out_shape=jax.ShapeDtypeStruct((_NG, _OUT_DIM), jnp.float32),
    )(target, emb, cw, cb, fw, fb)


def _head_body(pool_ref, xt_ref, fxw, fxb, f1w, f1b, f2w, f2b, ow, ob, o_ref):
    pooled = pool_ref[0] + pool_ref[1]
    xg = jnp.maximum(jnp.dot(pooled, fxw[...],
                             preferred_element_type=jnp.float32) + fxb[...], 0.0)
    xc = jnp.concatenate([xg, xt_ref[...]], axis=1)
    h1 = jnp.maximum(jnp.dot(xc, f1w[...],
                             preferred_element_type=jnp.float32) + f1b[...], 0.0)
    h2 = jnp.maximum(jnp.dot(h1, f2w[...],
                             preferred_element_type=jnp.float32) + f2b[...], 0.0)
    o_ref[...] = jnp.dot(h2, ow[...],
                         preferred_element_type=jnp.float32) + ob[...]


def _head(pool, xt, fxw, fxb, f1w, f1b, f2w, f2b, ow, ob):
    return pl.pallas_call(
        _head_body,
        out_shape=jax.ShapeDtypeStruct((_NG, 1), jnp.float32),
    )(pool, xt, fxw, fxb, f1w, f1b, f2w, f2b, ow, ob)


def _r2(b):
    return b.reshape(1, -1)


def kernel(x, edge_index, edge_attr, batch, target, params):
    p = params

    src = jnp.concatenate(
        [edge_index[0], jnp.zeros((_E_PAD - _E,), jnp.int32)]
    ).reshape(_E_PAD // 128, 128)
    dst = jnp.concatenate(
        [edge_index[1], jnp.zeros((_E_PAD - _E,), jnp.int32)]
    ).reshape(_E_PAD // 128, 128)
    ea_pad = jnp.pad(edge_attr, ((0, _E_PAD - _E), (0, 0)))

    elws, elbs = [], []
    w1s, b1s, w2s, b2s, gs, bbs = [], [], [], [], [], []
    for i in range(1, 6):
        w1, b1, w2, b2, elw, elb = p["c%d" % i]
        if i == 1:
            elw = jnp.pad(elw, ((0, 0), (0, 96 - _NFX)))
            elb = jnp.pad(elb, (0, 96 - _NFX))
            w1 = jnp.pad(w1, ((0, 96 - _NFX), (0, 0)))
        elws.append(elw)
        elbs.append(_r2(elb))
        w1s.append(w1)
        b1s.append(_r2(b1))
        w2s.append(w2)
        b2s.append(_r2(b2))
        gs.append(_r2(p["bn%d_g" % i] * _INV_STD))
        bbs.append(_r2(p["bn%d_b" % i]))

    ea_slabs = _eproj(ea_pad, p["ee_w"], _r2(p["ee_b"]), elws, elbs)

    x_pad = jnp.pad(x, ((0, 0), (0, 96 - _NFX)))
    x_tabs = [x_pad[:, 0:32], x_pad[:, 32:64], x_pad[:, 64:96]]

    aggs = [_edge_pass(x_tabs[j], ea_slabs[j], src, dst) for j in range(3)]
    h = _node_mlp1(x_pad, aggs, w1s[0], b1s[0], w2s[0], b2s[0], gs[0], bbs[0])
    for i in range(1, 5):
        agg = _edge_pass(h, ea_slabs[3 + i - 1], src, dst)
        h = _node_mlp(h, agg, w1s[i], b1s[i], w2s[i], b2s[i], gs[i], bbs[i])

    h_pad = jnp.pad(h, ((0, _N_PAD2 - _N), (0, 0)))
    batch_r = jnp.pad(batch, (0, _N_PAD2 - _N)).reshape(_N_PAD2 // 128, 128)
    pool = _pool_pass(h_pad, batch_r)

    xt = _protein(target, p["emb_xt"], p["conv_xt_w"], _r2(p["conv_xt_b"]),
                  p["fc_xt_w"], _r2(p["fc_xt_b"]))

    return _head(pool, xt, p["fcxd_w"], _r2(p["fcxd_b"]),
                 p["fc1_w"], _r2(p["fc1_b"]), p["fc2_w"], _r2(p["fc2_b"]),
                 p["out_w"], _r2(p["out_b"]))
